# baseline (speedup 1.0000x reference)
import functools

import jax
import jax.numpy as jnp
from jax.experimental import pallas as pl
from jax.experimental.pallas import tpu as pltpu


def _conv_kernel(x_ref, w_ref, shift_ref, maskl_ref, maskr_ref, o_ref,
                 *, w, m, kh, kw, pad):
    xb = x_ref[0].astype(jnp.bfloat16)
    c_in = xb.shape[0]
    zp = jnp.zeros((c_in, 2 * w), dtype=jnp.bfloat16)
    xf = jnp.concatenate([zp, xb, zp], axis=1)
    off = 2 * w
    taps = []
    for i in range(kh):
        for j in range(kw):
            s = off + (i - pad) * w + (j - pad)
            a = xf[:, s:s + m]
            if j == 0:
                a = a * maskl_ref[...]
            elif j == kw - 1:
                a = a * maskr_ref[...]
            taps.append(a)
    patches_t = jnp.concatenate(taps, axis=0)
    acc = jax.lax.dot_general(
        w_ref[...], patches_t, (((1,), (0,)), ((), ())),
        preferred_element_type=jnp.float32)
    o_ref[0] = jnp.maximum(acc + shift_ref[...], 0.0)


@jax.jit
def _basic_conv2d_opt(x_nchw, weight_oihw, gamma, beta, running_mean,
                      running_var):
    eps = 1e-3
    n, c_in, h, w = x_nchw.shape
    c_out, c_in_w, kh, kw = weight_oihw.shape
    assert c_in == c_in_w
    oh, ow = h, w
    m = oh * ow
    pad = 1

    x_flat = x_nchw.reshape(n, c_in, m)

    scale = gamma.astype(jnp.float32) / jnp.sqrt(
        running_var.astype(jnp.float32) + eps)
    shift = beta.astype(jnp.float32) - running_mean.astype(jnp.float32) * scale
    k_dim = kh * kw * c_in
    w_t = jnp.transpose(weight_oihw, (0, 2, 3, 1)).reshape(c_out, k_dim)
    w_t = (w_t.astype(jnp.float32) * scale[:, None]).astype(jnp.bfloat16)
    shift_col = shift.reshape(c_out, 1)

    col = jnp.arange(m, dtype=jnp.int32) % w
    mask_l = (col != 0).astype(jnp.bfloat16).reshape(1, m)
    mask_r = (col != w - 1).astype(jnp.bfloat16).reshape(1, m)

    out_flat = pl.pallas_call(
        functools.partial(_conv_kernel, w=w, m=m, kh=kh, kw=kw, pad=pad),
        out_shape=jax.ShapeDtypeStruct((n, c_out, m), jnp.float32),
        grid_spec=pltpu.PrefetchScalarGridSpec(
            num_scalar_prefetch=0,
            grid=(n,),
            in_specs=[
                pl.BlockSpec((1, c_in, m), lambda i: (i, 0, 0)),
                pl.BlockSpec((c_out, k_dim), lambda i: (0, 0)),
                pl.BlockSpec((c_out, 1), lambda i: (0, 0)),
                pl.BlockSpec((1, m), lambda i: (0, 0)),
                pl.BlockSpec((1, m), lambda i: (0, 0)),
            ],
            out_specs=pl.BlockSpec((1, c_out, m), lambda i: (i, 0, 0)),
        ),
        compiler_params=pltpu.CompilerParams(
            dimension_semantics=("parallel",),
            vmem_limit_bytes=64 * 1024 * 1024,
        ),
        cost_estimate=pl.CostEstimate(
            flops=2 * n * m * k_dim * c_out,
            transcendentals=0,
            bytes_accessed=n * (c_in * m * 4 + c_out * m * 4)
            + k_dim * c_out * 2,
        ),
    )(x_flat, w_t, shift_col, mask_l, mask_r)

    return out_flat.reshape(n, c_out, oh, ow)


def kernel(x_nchw, weight_oihw, gamma, beta, running_mean, running_var):
    return _basic_conv2d_opt(x_nchw, weight_oihw, gamma, beta, running_mean,
                             running_var)
